# baseline (speedup 1.0000x reference)
import functools

import jax
import jax.numpy as jnp
from jax import lax
from jax.experimental import pallas as pl
from jax.experimental.pallas import tpu as pltpu
from jax.experimental.pallas import tpu_sc as plsc

_N = 10000
_E = 320000
_NP = 10240
_NC, _NS = 2, 16
_NW = _NC * _NS
_CE = 128
_EP = 327680
_EPW = _EP // _NW
_NCHUNK = _EPW // _CE


def _vsc_mesh():
    return plsc.VectorSubcoreMesh(core_axis_name="c", subcore_axis_name="s",
                                  num_cores=_NC, num_subcores=_NS)


@functools.partial(
    pl.kernel,
    mesh=_vsc_mesh(),
    compiler_params=pltpu.CompilerParams(use_tc_tiling_on_sc=False),
    out_type=jax.ShapeDtypeStruct((_NC, _NP, 16), jnp.float32),
    scratch_types=[
        pltpu.VMEM((_NCHUNK, _CE), jnp.int32),
        pltpu.VMEM((_CE, 16), jnp.float32),
        pltpu.VMEM((16, 16), jnp.float32),
        pltpu.VMEM_SHARED((_NP, 16), jnp.float32),
        pltpu.SemaphoreType.DMA,
    ],
)
def _deg_kernel(row_hbm, out_hbm, idx_v, ones_v, zb_v, sacc, zsem):
    cid = lax.axis_index("c")
    sid = lax.axis_index("s")
    wid = cid * _NS + sid
    zeros16 = jnp.zeros((16,), jnp.float32)
    ones16 = jnp.ones((16,), jnp.float32)
    rows_per_tile = _NP // 16

    for i in range(16):
        zb_v[i, :] = zeros16

    def _fill(i, _):
        ones_v[i, :] = ones16
        return 0

    lax.fori_loop(0, _CE, _fill, 0)

    zd = [pltpu.async_copy(zb_v,
                           sacc.at[pl.ds(sid * rows_per_tile + j * 16, 16)],
                           zsem)
          for j in range(rows_per_tile // 16)]
    for d in zd:
        d.wait()
    plsc.subcore_barrier()

    pltpu.sync_copy(row_hbm.at[pl.ds(wid * _NCHUNK, _NCHUNK)], idx_v)

    def _chunk(e, _):
        pltpu.sync_copy(ones_v, sacc.at[idx_v.at[e]], add=True)
        return 0

    lax.fori_loop(0, _NCHUNK, _chunk, 0)
    plsc.subcore_barrier()

    pltpu.sync_copy(sacc.at[pl.ds(sid * rows_per_tile, rows_per_tile)],
                    out_hbm.at[cid, pl.ds(sid * rows_per_tile, rows_per_tile)])


def _make_lap_kernel(F, ce=_CE):
    rows_per_tile = _NP // 16
    nchunk = _EPW // ce

    @functools.partial(
        pl.kernel,
        mesh=_vsc_mesh(),
        compiler_params=pltpu.CompilerParams(use_tc_tiling_on_sc=False),
        out_type=jax.ShapeDtypeStruct((_NC, _NP, F), jnp.float32),
        scratch_types=[
            pltpu.VMEM((nchunk // 2, ce), jnp.int32),
            pltpu.VMEM((nchunk // 2, ce), jnp.int32),
            pltpu.VMEM((ce, F), jnp.float32),
            pltpu.VMEM((ce, F), jnp.float32),
            pltpu.VMEM((16, F), jnp.float32),
            pltpu.VMEM_SHARED((_NP, F), jnp.float32),
            pltpu.SemaphoreType.DMA,
            pltpu.SemaphoreType.DMA,
        ],
    )
    def k(tab_hbm, row_hbm, col_hbm, out_hbm,
          idx_r, idx_c, buf0, buf1, zb_v, sacc, sem0, sem1):
        cid = lax.axis_index("c")
        sid = lax.axis_index("s")
        wid = cid * _NS + sid
        zeros16 = jnp.zeros((16,), jnp.float32)
        half = nchunk // 2

        for i in range(16):
            for c in range(F // 16):
                zb_v[i, pl.ds(c * 16, 16)] = zeros16
        zd = [pltpu.async_copy(zb_v,
                               sacc.at[pl.ds(sid * rows_per_tile + j * 16, 16)],
                               sem0)
              for j in range(rows_per_tile // 16)]
        for d in zd:
            d.wait()
        plsc.subcore_barrier()

        def _super(h, _):
            pltpu.sync_copy(row_hbm.at[pl.ds(wid * nchunk + h * half, half)],
                            idx_r)
            pltpu.sync_copy(col_hbm.at[pl.ds(wid * nchunk + h * half, half)],
                            idx_c)

            pltpu.async_copy(tab_hbm.at[idx_r.at[0]], buf0, sem0)

            def _pair(j, _):
                e = 2 * j
                pltpu.async_copy(tab_hbm.at[idx_r.at[e + 1]], buf1, sem1)
                pltpu.make_async_copy(tab_hbm.at[idx_r.at[e]], buf0, sem0).wait()
                pltpu.sync_copy(buf0, sacc.at[idx_c.at[e]], add=True)

                @pl.when(j < half // 2 - 1)
                def _():
                    pltpu.async_copy(tab_hbm.at[idx_r.at[e + 2]], buf0, sem0)

                pltpu.make_async_copy(tab_hbm.at[idx_r.at[e + 1]], buf1, sem1).wait()
                pltpu.sync_copy(buf1, sacc.at[idx_c.at[e + 1]], add=True)
                return 0

            lax.fori_loop(0, half // 2, _pair, 0)
            return 0

        lax.fori_loop(0, 2, _super, 0)
        plsc.subcore_barrier()

        pltpu.sync_copy(sacc.at[pl.ds(sid * rows_per_tile, rows_per_tile)],
                        out_hbm.at[cid, pl.ds(sid * rows_per_tile, rows_per_tile)])

    return k


_LAP_CE = {64: 512, 32: 512}
_lap_kernels = {F: _make_lap_kernel(F, _LAP_CE[F]) for F in (64, 32)}
_CE0 = 512


def _make_lap_split():
    F = 64
    ept = _EP // _NS
    nchunk = ept // _CE0
    rows_per_tile = _NP // 16

    @functools.partial(
        pl.kernel,
        mesh=_vsc_mesh(),
        compiler_params=pltpu.CompilerParams(use_tc_tiling_on_sc=False),
        out_type=jax.ShapeDtypeStruct((_NC, _NP, F), jnp.float32),
        scratch_types=[
            pltpu.VMEM((nchunk // 2, _CE0), jnp.int32),
            pltpu.VMEM((nchunk // 2, _CE0), jnp.int32),
            pltpu.VMEM((_CE0, F), jnp.float32),
            pltpu.VMEM((_CE0, F), jnp.float32),
            pltpu.VMEM((16, F), jnp.float32),
            pltpu.VMEM_SHARED((_NP, F), jnp.float32),
            pltpu.SemaphoreType.DMA,
            pltpu.SemaphoreType.DMA,
        ],
    )
    def k(tab_lo, tab_hi, row_hbm, col_hbm, out_hbm,
          idx_r, idx_c, buf0, buf1, zb_v, sacc, sem0, sem1):
        cid = lax.axis_index("c")
        sid = lax.axis_index("s")
        zeros16 = jnp.zeros((16,), jnp.float32)
        half = nchunk // 2

        for i in range(16):
            for c in range(F // 16):
                zb_v[i, pl.ds(c * 16, 16)] = zeros16
        zd = [pltpu.async_copy(zb_v,
                               sacc.at[pl.ds(sid * rows_per_tile + j * 16, 16)],
                               sem0)
              for j in range(rows_per_tile // 16)]
        for d in zd:
            d.wait()
        plsc.subcore_barrier()

        def _run(tab_hbm):
            def _super(h, _):
                pltpu.sync_copy(row_hbm.at[pl.ds(sid * nchunk + h * half, half)],
                                idx_r)
                pltpu.sync_copy(col_hbm.at[pl.ds(sid * nchunk + h * half, half)],
                                idx_c)
                pltpu.async_copy(tab_hbm.at[idx_r.at[0]], buf0, sem0)

                def _pair(j, _):
                    e = 2 * j
                    pltpu.async_copy(tab_hbm.at[idx_r.at[e + 1]], buf1, sem1)
                    pltpu.make_async_copy(tab_hbm.at[idx_r.at[e]], buf0,
                                          sem0).wait()
                    pltpu.sync_copy(buf0, sacc.at[idx_c.at[e]], add=True)

                    @pl.when(j < half // 2 - 1)
                    def _():
                        pltpu.async_copy(tab_hbm.at[idx_r.at[e + 2]], buf0, sem0)

                    pltpu.make_async_copy(tab_hbm.at[idx_r.at[e + 1]], buf1,
                                          sem1).wait()
                    pltpu.sync_copy(buf1, sacc.at[idx_c.at[e + 1]], add=True)
                    return 0

                lax.fori_loop(0, half // 2, _pair, 0)
                return 0

            lax.fori_loop(0, 2, _super, 0)

        @pl.when(cid == 0)
        def _():
            _run(tab_lo)

        @pl.when(cid == 1)
        def _():
            _run(tab_hi)

        plsc.subcore_barrier()
        pltpu.sync_copy(sacc.at[pl.ds(sid * rows_per_tile, rows_per_tile)],
                        out_hbm.at[cid, pl.ds(sid * rows_per_tile, rows_per_tile)])

    return k


_lap_split = _make_lap_split()


_BLK = 2048
_GRID = _NP // _BLK


def _tca_body(degp_ref, x_ref, dis_ref, lo_ref, hi_ref):
    deg = degp_ref[0] + degp_ref[1]
    safe = jnp.maximum(deg, 1.0)
    dis = jnp.where(deg > 0, lax.rsqrt(safe), 0.0)
    dis_ref[...] = dis
    xs = dis * x_ref[...]
    lo_ref[...] = xs[:, :64]
    hi_ref[...] = xs[:, 64:]


_tca = pl.pallas_call(
    _tca_body,
    grid=(_GRID,),
    in_specs=[
        pl.BlockSpec((2, _BLK, 1), lambda i: (0, i, 0)),
        pl.BlockSpec((_BLK, 128), lambda i: (i, 0)),
    ],
    out_specs=[
        pl.BlockSpec((_BLK, 1), lambda i: (i, 0)),
        pl.BlockSpec((_BLK, 64), lambda i: (i, 0)),
        pl.BlockSpec((_BLK, 64), lambda i: (i, 0)),
    ],
    out_shape=[
        jax.ShapeDtypeStruct((_NP, 1), jnp.float32),
        jax.ShapeDtypeStruct((_NP, 64), jnp.float32),
        jax.ShapeDtypeStruct((_NP, 64), jnp.float32),
    ],
)


def _lrelu(v):
    return jnp.where(v > 0, v, 0.1 * v)


def _make_gate_stage(F_in, F_acc, F_next, last, split_table):
    F = F_in

    def body(h_ref, acc_ref, dis_ref,
             w0i, w1ai, w1bi, bi, w0c, w1ac, w1bc, bc,
             w0o, w1ao, w1bo, bo, wco, wl, bl,
             *out_refs):
        h = h_ref[...]
        dis = dis_ref[...]
        ya = (-dis) * acc_ref[0]
        yb = (-dis) * acc_ref[1]

        def pre(w0, w1a, w1b, b):
            return (jnp.dot(h, w0[...], preferred_element_type=jnp.float32)
                    + jnp.dot(ya, w1a[...], preferred_element_type=jnp.float32)
                    + jnp.dot(yb, w1b[...], preferred_element_type=jnp.float32)
                    + b[...])

        gi = jax.nn.sigmoid(pre(w0i, w1ai, w1bi, bi))
        gt = jnp.tanh(pre(w0c, w1ac, w1bc, bc))
        gc = gi * gt
        go = jax.nn.sigmoid(pre(w0o, w1ao, w1bo, bo) + wco[...] * gc)
        hh = _lrelu(go * jnp.tanh(gc))
        z = jnp.dot(hh, wl[...], preferred_element_type=jnp.float32) + bl[...]
        if last:
            out_refs[0][...] = z
            return
        z = _lrelu(z)
        out_refs[0][...] = z
        t = dis * z
        if split_table:
            out_refs[1][...] = t[:, :F_next // 2]
            out_refs[2][...] = t[:, F_next // 2:]
        else:
            out_refs[1][...] = t

    wspec = lambda a, b: pl.BlockSpec((a, b), lambda i: (0, 0))
    in_specs = [
        pl.BlockSpec((_BLK, F_in), lambda i: (i, 0)),
        pl.BlockSpec((2, _BLK, F_acc), lambda i: (0, i, 0)),
        pl.BlockSpec((_BLK, 1), lambda i: (i, 0)),
    ]
    for _ in range(3):
        in_specs += [wspec(F_in, F), wspec(F_acc, F), wspec(F_acc, F),
                     wspec(1, F)]
    in_specs += [wspec(1, F), wspec(F, F_next), wspec(1, F_next)]

    out_specs = [pl.BlockSpec((_BLK, F_next), lambda i: (i, 0))]
    out_shape = [jax.ShapeDtypeStruct((_NP, F_next), jnp.float32)]
    if not last:
        tw = F_next // 2 if split_table else F_next
        n_tab = 2 if split_table else 1
        out_specs += [pl.BlockSpec((_BLK, tw), lambda i: (i, 0))] * n_tab
        out_shape += [jax.ShapeDtypeStruct((_NP, tw), jnp.float32)] * n_tab
    return pl.pallas_call(body, grid=(_GRID,), in_specs=in_specs,
                          out_specs=out_specs, out_shape=out_shape)


_gate_stages = [
    _make_gate_stage(128, 64, 64, False, False),
    _make_gate_stage(64, 64, 32, False, False),
    _make_gate_stage(32, 32, 128, True, False),
]


def _gate_params(p, lin, split_acc):
    out = []
    for g in ("i", "c", "o"):
        cx, ch = p["conv_x_" + g], p["conv_h_" + g]
        w1 = cx["W"][1]
        if split_acc:
            w1a, w1b = w1[:w1.shape[0] // 2], w1[w1.shape[0] // 2:]
        else:
            w1a = w1b = w1
        b = (cx["b"] + ch["b"])[None, :] + p["b_" + g]
        out += [cx["W"][0], w1a, w1b, b]
    out += [p["w_c_o"], lin["W"], lin["b"][None, :]]
    return out


def kernel(x, edge_index, params):
    pad = _N + (jnp.arange(_EP - _E, dtype=jnp.int32) % (_NP - _N))
    rowp = jnp.concatenate([edge_index[0], pad])
    colp = jnp.concatenate([edge_index[1], pad])
    edges2d = {ce: (rowp.reshape(-1, ce), colp.reshape(-1, ce))
               for ce in set(_LAP_CE.values()) | {_CE}}

    degp = _deg_kernel(edges2d[_CE][0])
    degp = degp[:, :, :1]

    x_pad = jnp.pad(x, ((0, _NP - _N), (0, 0)))
    dis, tab_lo, tab_hi = _tca(degp, x_pad)

    r2, c2 = edges2d[_CE0]
    accp = _lap_split(tab_lo, tab_hi, r2, c2)
    args = _gate_params(params["lstm0"], params["lin0"], True)
    h, tab = _gate_stages[0](x_pad, accp, dis, *args)

    for li, F in ((1, 64), (2, 32)):
        r2, c2 = edges2d[_LAP_CE[F]]
        accp = _lap_kernels[F](tab, r2, c2)
        args = _gate_params(params["lstm" + str(li)],
                            params["lin" + str(li)], False)
        outs = _gate_stages[li](h, accp, dis, *args)
        h = outs[0]
        if li < 2:
            tab = outs[1]
    return h[:_N]


# speedup vs baseline: 1.0461x; 1.0461x over previous
import functools

import jax
import jax.numpy as jnp
from jax import lax
from jax.experimental import pallas as pl
from jax.experimental.pallas import tpu as pltpu
from jax.experimental.pallas import tpu_sc as plsc

_N = 10000
_E = 320000
_NP = 10240
_NC, _NS = 2, 16
_NW = _NC * _NS
_CE = 128
_EP = 327680
_EPW = _EP // _NW
_NCHUNK = _EPW // _CE


def _vsc_mesh():
    return plsc.VectorSubcoreMesh(core_axis_name="c", subcore_axis_name="s",
                                  num_cores=_NC, num_subcores=_NS)


@functools.partial(
    pl.kernel,
    mesh=_vsc_mesh(),
    compiler_params=pltpu.CompilerParams(use_tc_tiling_on_sc=False),
    out_type=jax.ShapeDtypeStruct((_NC, _NP, 16), jnp.float32),
    scratch_types=[
        pltpu.VMEM((_NCHUNK, _CE), jnp.int32),
        pltpu.VMEM((_CE, 16), jnp.float32),
        pltpu.VMEM((16, 16), jnp.float32),
        pltpu.VMEM_SHARED((_NP, 16), jnp.float32),
        pltpu.SemaphoreType.DMA,
    ],
)
def _deg_kernel(row_hbm, out_hbm, idx_v, ones_v, zb_v, sacc, zsem):
    cid = lax.axis_index("c")
    sid = lax.axis_index("s")
    wid = cid * _NS + sid
    zeros16 = jnp.zeros((16,), jnp.float32)
    ones16 = jnp.ones((16,), jnp.float32)
    rows_per_tile = _NP // 16

    for i in range(16):
        zb_v[i, :] = zeros16

    def _fill(i, _):
        ones_v[i, :] = ones16
        return 0

    lax.fori_loop(0, _CE, _fill, 0)

    zd = [pltpu.async_copy(zb_v,
                           sacc.at[pl.ds(sid * rows_per_tile + j * 16, 16)],
                           zsem)
          for j in range(rows_per_tile // 16)]
    for d in zd:
        d.wait()
    plsc.subcore_barrier()

    pltpu.sync_copy(row_hbm.at[pl.ds(wid * _NCHUNK, _NCHUNK)], idx_v)

    def _chunk(e, _):
        pltpu.sync_copy(ones_v, sacc.at[idx_v.at[e]], add=True)
        return 0

    lax.fori_loop(0, _NCHUNK, _chunk, 0)
    plsc.subcore_barrier()

    pltpu.sync_copy(sacc.at[pl.ds(sid * rows_per_tile, rows_per_tile)],
                    out_hbm.at[cid, pl.ds(sid * rows_per_tile, rows_per_tile)])


def _make_lap_kernel(F, ce=_CE):
    rows_per_tile = _NP // 16
    nchunk = _EPW // ce

    @functools.partial(
        pl.kernel,
        mesh=_vsc_mesh(),
        compiler_params=pltpu.CompilerParams(use_tc_tiling_on_sc=False),
        out_type=jax.ShapeDtypeStruct((_NC, _NP, F), jnp.float32),
        scratch_types=[
            pltpu.VMEM((nchunk // 2, ce), jnp.int32),
            pltpu.VMEM((nchunk // 2, ce), jnp.int32),
            pltpu.VMEM((ce, F), jnp.float32),
            pltpu.VMEM((ce, F), jnp.float32),
            pltpu.VMEM((16, F), jnp.float32),
            pltpu.VMEM_SHARED((_NP, F), jnp.float32),
            pltpu.SemaphoreType.DMA,
            pltpu.SemaphoreType.DMA,
        ],
    )
    def k(tab_hbm, row_hbm, col_hbm, out_hbm,
          idx_r, idx_c, buf0, buf1, zb_v, sacc, sem0, sem1):
        cid = lax.axis_index("c")
        sid = lax.axis_index("s")
        wid = cid * _NS + sid
        zeros16 = jnp.zeros((16,), jnp.float32)
        half = nchunk // 2

        for i in range(16):
            for c in range(F // 16):
                zb_v[i, pl.ds(c * 16, 16)] = zeros16
        zd = [pltpu.async_copy(zb_v,
                               sacc.at[pl.ds(sid * rows_per_tile + j * 16, 16)],
                               sem0)
              for j in range(rows_per_tile // 16)]
        for d in zd:
            d.wait()
        plsc.subcore_barrier()

        def _super(h, _):
            pltpu.sync_copy(row_hbm.at[pl.ds(wid * nchunk + h * half, half)],
                            idx_r)
            pltpu.sync_copy(col_hbm.at[pl.ds(wid * nchunk + h * half, half)],
                            idx_c)

            pltpu.async_copy(tab_hbm.at[idx_r.at[0]], buf0, sem0)

            def _pair(j, _):
                e = 2 * j
                pltpu.async_copy(tab_hbm.at[idx_r.at[e + 1]], buf1, sem1)
                pltpu.make_async_copy(tab_hbm.at[idx_r.at[e]], buf0, sem0).wait()
                pltpu.sync_copy(buf0, sacc.at[idx_c.at[e]], add=True)

                @pl.when(j < half // 2 - 1)
                def _():
                    pltpu.async_copy(tab_hbm.at[idx_r.at[e + 2]], buf0, sem0)

                pltpu.make_async_copy(tab_hbm.at[idx_r.at[e + 1]], buf1, sem1).wait()
                pltpu.sync_copy(buf1, sacc.at[idx_c.at[e + 1]], add=True)
                return 0

            lax.fori_loop(0, half // 2, _pair, 0)
            return 0

        lax.fori_loop(0, 2, _super, 0)
        plsc.subcore_barrier()

        pltpu.sync_copy(sacc.at[pl.ds(sid * rows_per_tile, rows_per_tile)],
                        out_hbm.at[cid, pl.ds(sid * rows_per_tile, rows_per_tile)])

    return k


_LAP_CE = {128: 128, 64: 512, 32: 512}
_lap_kernels = {F: _make_lap_kernel(F, _LAP_CE[F]) for F in (128, 64, 32)}


_BLK = 2048
_GRID = _NP // _BLK


def _tca_body(degp_ref, x_ref, dis_ref, xs_ref):
    deg = degp_ref[0] + degp_ref[1]
    safe = jnp.maximum(deg, 1.0)
    dis = jnp.where(deg > 0, lax.rsqrt(safe), 0.0)
    dis_ref[...] = dis
    xs_ref[...] = dis * x_ref[...]


_tca = pl.pallas_call(
    _tca_body,
    grid=(_GRID,),
    in_specs=[
        pl.BlockSpec((2, _BLK, 1), lambda i: (0, i, 0)),
        pl.BlockSpec((_BLK, 128), lambda i: (i, 0)),
    ],
    out_specs=[
        pl.BlockSpec((_BLK, 1), lambda i: (i, 0)),
        pl.BlockSpec((_BLK, 128), lambda i: (i, 0)),
    ],
    out_shape=[
        jax.ShapeDtypeStruct((_NP, 1), jnp.float32),
        jax.ShapeDtypeStruct((_NP, 128), jnp.float32),
    ],
)


def _lrelu(v):
    return jnp.where(v > 0, v, 0.1 * v)


def _make_gate_stage(F_in, F_acc, F_next, last, split_table):
    F = F_in

    def body(h_ref, acc_ref, dis_ref,
             w0i, w1ai, w1bi, bi, w0c, w1ac, w1bc, bc,
             w0o, w1ao, w1bo, bo, wco, wl, bl,
             *out_refs):
        h = h_ref[...]
        dis = dis_ref[...]
        ya = (-dis) * acc_ref[0]
        yb = (-dis) * acc_ref[1]

        def pre(w0, w1a, w1b, b):
            return (jnp.dot(h, w0[...], preferred_element_type=jnp.float32)
                    + jnp.dot(ya, w1a[...], preferred_element_type=jnp.float32)
                    + jnp.dot(yb, w1b[...], preferred_element_type=jnp.float32)
                    + b[...])

        gi = jax.nn.sigmoid(pre(w0i, w1ai, w1bi, bi))
        gt = jnp.tanh(pre(w0c, w1ac, w1bc, bc))
        gc = gi * gt
        go = jax.nn.sigmoid(pre(w0o, w1ao, w1bo, bo) + wco[...] * gc)
        hh = _lrelu(go * jnp.tanh(gc))
        z = jnp.dot(hh, wl[...], preferred_element_type=jnp.float32) + bl[...]
        if last:
            out_refs[0][...] = z
            return
        z = _lrelu(z)
        out_refs[0][...] = z
        t = dis * z
        if split_table:
            out_refs[1][...] = t[:, :F_next // 2]
            out_refs[2][...] = t[:, F_next // 2:]
        else:
            out_refs[1][...] = t

    wspec = lambda a, b: pl.BlockSpec((a, b), lambda i: (0, 0))
    in_specs = [
        pl.BlockSpec((_BLK, F_in), lambda i: (i, 0)),
        pl.BlockSpec((2, _BLK, F_acc), lambda i: (0, i, 0)),
        pl.BlockSpec((_BLK, 1), lambda i: (i, 0)),
    ]
    for _ in range(3):
        in_specs += [wspec(F_in, F), wspec(F_acc, F), wspec(F_acc, F),
                     wspec(1, F)]
    in_specs += [wspec(1, F), wspec(F, F_next), wspec(1, F_next)]

    out_specs = [pl.BlockSpec((_BLK, F_next), lambda i: (i, 0))]
    out_shape = [jax.ShapeDtypeStruct((_NP, F_next), jnp.float32)]
    if not last:
        tw = F_next // 2 if split_table else F_next
        n_tab = 2 if split_table else 1
        out_specs += [pl.BlockSpec((_BLK, tw), lambda i: (i, 0))] * n_tab
        out_shape += [jax.ShapeDtypeStruct((_NP, tw), jnp.float32)] * n_tab
    return pl.pallas_call(body, grid=(_GRID,), in_specs=in_specs,
                          out_specs=out_specs, out_shape=out_shape)


_gate_stages = [
    _make_gate_stage(128, 128, 64, False, False),
    _make_gate_stage(64, 64, 32, False, False),
    _make_gate_stage(32, 32, 128, True, False),
]


def _gate_params(p, lin, split_acc):
    out = []
    for g in ("i", "c", "o"):
        cx, ch = p["conv_x_" + g], p["conv_h_" + g]
        w1 = cx["W"][1]
        if split_acc:
            w1a, w1b = w1[:w1.shape[0] // 2], w1[w1.shape[0] // 2:]
        else:
            w1a = w1b = w1
        b = (cx["b"] + ch["b"])[None, :] + p["b_" + g]
        out += [cx["W"][0], w1a, w1b, b]
    out += [p["w_c_o"], lin["W"], lin["b"][None, :]]
    return out


def kernel(x, edge_index, params):
    pad = _N + (jnp.arange(_EP - _E, dtype=jnp.int32) % (_NP - _N))
    rowp = jnp.concatenate([edge_index[0], pad])
    colp = jnp.concatenate([edge_index[1], pad])
    edges2d = {ce: (rowp.reshape(-1, ce), colp.reshape(-1, ce))
               for ce in set(_LAP_CE.values()) | {_CE}}

    degp = _deg_kernel(edges2d[_CE][0])
    degp = degp[:, :, :1]

    x_pad = jnp.pad(x, ((0, _NP - _N), (0, 0)))
    dis, tab = _tca(degp, x_pad)

    h = x_pad
    for li, F in ((0, 128), (1, 64), (2, 32)):
        r2, c2 = edges2d[_LAP_CE[F]]
        accp = _lap_kernels[F](tab, r2, c2)
        args = _gate_params(params["lstm" + str(li)],
                            params["lin" + str(li)], False)
        outs = _gate_stages[li](h, accp, dis, *args)
        h = outs[0]
        if li < 2:
            tab = outs[1]
    return h[:_N]


# speedup vs baseline: 1.0776x; 1.0301x over previous
import functools

import jax
import jax.numpy as jnp
from jax import lax
from jax.experimental import pallas as pl
from jax.experimental.pallas import tpu as pltpu
from jax.experimental.pallas import tpu_sc as plsc

_N = 10000
_E = 320000
_NP = 10240
_NC, _NS = 2, 16
_NW = _NC * _NS
_CE = 128
_EP = 327680
_EPW = _EP // _NW
_NCHUNK = _EPW // _CE


def _vsc_mesh():
    return plsc.VectorSubcoreMesh(core_axis_name="c", subcore_axis_name="s",
                                  num_cores=_NC, num_subcores=_NS)


_DCE = 512
_DNCHUNK = _EPW // _DCE


@functools.partial(
    pl.kernel,
    mesh=_vsc_mesh(),
    compiler_params=pltpu.CompilerParams(use_tc_tiling_on_sc=False),
    out_type=jax.ShapeDtypeStruct((_NC, _NP, 16), jnp.float32),
    scratch_types=[
        pltpu.VMEM((_DNCHUNK, _DCE), jnp.int32),
        pltpu.VMEM((_DCE, 16), jnp.float32),
        pltpu.VMEM((16, 16), jnp.float32),
        pltpu.VMEM_SHARED((_NP, 16), jnp.float32),
        pltpu.SemaphoreType.DMA,
    ],
)
def _deg_kernel(row_hbm, out_hbm, idx_v, ones_v, zb_v, sacc, zsem):
    cid = lax.axis_index("c")
    sid = lax.axis_index("s")
    wid = cid * _NS + sid
    zeros16 = jnp.zeros((16,), jnp.float32)
    ones16 = jnp.ones((16,), jnp.float32)
    rows_per_tile = _NP // 16

    for i in range(16):
        zb_v[i, :] = zeros16

    def _fill(i, _):
        ones_v[i, :] = ones16
        return 0

    lax.fori_loop(0, _DCE, _fill, 0)

    zd = [pltpu.async_copy(zb_v,
                           sacc.at[pl.ds(sid * rows_per_tile + j * 16, 16)],
                           zsem)
          for j in range(rows_per_tile // 16)]
    for d in zd:
        d.wait()
    plsc.subcore_barrier()

    pltpu.sync_copy(row_hbm.at[pl.ds(wid * _DNCHUNK, _DNCHUNK)], idx_v)

    def _chunk(e, _):
        pltpu.sync_copy(ones_v, sacc.at[idx_v.at[e]], add=True)
        return 0

    lax.fori_loop(0, _DNCHUNK, _chunk, 0)
    plsc.subcore_barrier()

    pltpu.sync_copy(sacc.at[pl.ds(sid * rows_per_tile, rows_per_tile)],
                    out_hbm.at[cid, pl.ds(sid * rows_per_tile, rows_per_tile)])


def _make_lap_kernel(F, ce=_CE):
    rows_per_tile = _NP // 16
    nchunk = _EPW // ce

    @functools.partial(
        pl.kernel,
        mesh=_vsc_mesh(),
        compiler_params=pltpu.CompilerParams(use_tc_tiling_on_sc=False),
        out_type=jax.ShapeDtypeStruct((_NC, _NP, F), jnp.float32),
        scratch_types=[
            pltpu.VMEM((nchunk // 2, ce), jnp.int32),
            pltpu.VMEM((nchunk // 2, ce), jnp.int32),
            pltpu.VMEM((ce, F), jnp.float32),
            pltpu.VMEM((ce, F), jnp.float32),
            pltpu.VMEM((16, F), jnp.float32),
            pltpu.VMEM_SHARED((_NP, F), jnp.float32),
            pltpu.SemaphoreType.DMA,
            pltpu.SemaphoreType.DMA,
        ],
    )
    def k(tab_hbm, row_hbm, col_hbm, out_hbm,
          idx_r, idx_c, buf0, buf1, zb_v, sacc, sem0, sem1):
        cid = lax.axis_index("c")
        sid = lax.axis_index("s")
        wid = cid * _NS + sid
        zeros16 = jnp.zeros((16,), jnp.float32)
        half = nchunk // 2

        for i in range(16):
            for c in range(F // 16):
                zb_v[i, pl.ds(c * 16, 16)] = zeros16
        zd = [pltpu.async_copy(zb_v,
                               sacc.at[pl.ds(sid * rows_per_tile + j * 16, 16)],
                               sem0)
              for j in range(rows_per_tile // 16)]
        for d in zd:
            d.wait()
        plsc.subcore_barrier()

        def _super(h, _):
            pltpu.sync_copy(row_hbm.at[pl.ds(wid * nchunk + h * half, half)],
                            idx_r)
            pltpu.sync_copy(col_hbm.at[pl.ds(wid * nchunk + h * half, half)],
                            idx_c)

            pltpu.async_copy(tab_hbm.at[idx_r.at[0]], buf0, sem0)

            def _pair(j, _):
                e = 2 * j
                pltpu.async_copy(tab_hbm.at[idx_r.at[e + 1]], buf1, sem1)
                pltpu.make_async_copy(tab_hbm.at[idx_r.at[e]], buf0, sem0).wait()
                pltpu.sync_copy(buf0, sacc.at[idx_c.at[e]], add=True)

                @pl.when(j < half // 2 - 1)
                def _():
                    pltpu.async_copy(tab_hbm.at[idx_r.at[e + 2]], buf0, sem0)

                pltpu.make_async_copy(tab_hbm.at[idx_r.at[e + 1]], buf1, sem1).wait()
                pltpu.sync_copy(buf1, sacc.at[idx_c.at[e + 1]], add=True)
                return 0

            lax.fori_loop(0, half // 2, _pair, 0)
            return 0

        lax.fori_loop(0, 2, _super, 0)
        plsc.subcore_barrier()

        pltpu.sync_copy(sacc.at[pl.ds(sid * rows_per_tile, rows_per_tile)],
                        out_hbm.at[cid, pl.ds(sid * rows_per_tile, rows_per_tile)])

    return k


_LAP_CE = {128: 128, 64: 512, 32: 512}
_lap_kernels = {F: _make_lap_kernel(F, _LAP_CE[F]) for F in (128, 64, 32)}


_BLK = 2048
_GRID = _NP // _BLK


def _tca_body(degp_ref, x_ref, dis_ref, xs_ref):
    deg = degp_ref[0] + degp_ref[1]
    safe = jnp.maximum(deg, 1.0)
    dis = jnp.where(deg > 0, lax.rsqrt(safe), 0.0)
    dis_ref[...] = dis
    xs_ref[...] = dis * x_ref[...]


_tca = pl.pallas_call(
    _tca_body,
    grid=(_GRID,),
    in_specs=[
        pl.BlockSpec((2, _BLK, 1), lambda i: (0, i, 0)),
        pl.BlockSpec((_BLK, 128), lambda i: (i, 0)),
    ],
    out_specs=[
        pl.BlockSpec((_BLK, 1), lambda i: (i, 0)),
        pl.BlockSpec((_BLK, 128), lambda i: (i, 0)),
    ],
    out_shape=[
        jax.ShapeDtypeStruct((_NP, 1), jnp.float32),
        jax.ShapeDtypeStruct((_NP, 128), jnp.float32),
    ],
)


def _lrelu(v):
    return jnp.where(v > 0, v, 0.1 * v)


def _make_gate_stage(F_in, F_next, last):
    F = F_in
    blk = 2000 if last else _BLK
    n_rows = _N if last else _NP
    grid = n_rows // blk

    def body(h_ref, acc_ref, dis_ref,
             w0i, w1i, bi, w0c, w1c, bc, w0o, w1o, bo, wco, wl, bl,
             *out_refs):
        h = h_ref[...]
        dis = dis_ref[...]
        y = (-dis) * (acc_ref[0] + acc_ref[1])

        def pre(w0, w1, b):
            return (jnp.dot(h, w0[...], preferred_element_type=jnp.float32)
                    + jnp.dot(y, w1[...], preferred_element_type=jnp.float32)
                    + b[...])

        gi = jax.nn.sigmoid(pre(w0i, w1i, bi))
        gt = jnp.tanh(pre(w0c, w1c, bc))
        gc = gi * gt
        go = jax.nn.sigmoid(pre(w0o, w1o, bo) + wco[...] * gc)
        hh = _lrelu(go * jnp.tanh(gc))
        z = jnp.dot(hh, wl[...], preferred_element_type=jnp.float32) + bl[...]
        if last:
            out_refs[0][...] = z
        else:
            z = _lrelu(z)
            out_refs[0][...] = z
            out_refs[1][...] = dis * z

    wspec = lambda a, b: pl.BlockSpec((a, b), lambda i: (0, 0))
    in_specs = [
        pl.BlockSpec((blk, F_in), lambda i: (i, 0)),
        pl.BlockSpec((2, blk, F), lambda i: (0, i, 0)),
        pl.BlockSpec((blk, 1), lambda i: (i, 0)),
        wspec(F_in, F), wspec(F, F), wspec(1, F),
        wspec(F_in, F), wspec(F, F), wspec(1, F),
        wspec(F_in, F), wspec(F, F), wspec(1, F),
        wspec(1, F),
        wspec(F, F_next), wspec(1, F_next),
    ]
    out_specs = [pl.BlockSpec((blk, F_next), lambda i: (i, 0))]
    out_shape = [jax.ShapeDtypeStruct((n_rows, F_next), jnp.float32)]
    if not last:
        out_specs += [pl.BlockSpec((blk, F_next), lambda i: (i, 0))]
        out_shape += [jax.ShapeDtypeStruct((n_rows, F_next), jnp.float32)]
    return pl.pallas_call(body, grid=(grid,), in_specs=in_specs,
                          out_specs=out_specs, out_shape=out_shape)


_gate_stages = [
    _make_gate_stage(128, 64, False),
    _make_gate_stage(64, 32, False),
    _make_gate_stage(32, 128, True),
]


def _gate_params(p, lin):
    out = []
    for g in ("i", "c", "o"):
        cx, ch = p["conv_x_" + g], p["conv_h_" + g]
        b = (cx["b"] + ch["b"])[None, :] + p["b_" + g]
        out += [cx["W"][0], cx["W"][1], b]
    out += [p["w_c_o"], lin["W"], lin["b"][None, :]]
    return out


def kernel(x, edge_index, params):
    pad = _N + (jnp.arange(_EP - _E, dtype=jnp.int32) % (_NP - _N))
    rowp = jnp.concatenate([edge_index[0], pad])
    colp = jnp.concatenate([edge_index[1], pad])
    edges2d = {ce: (rowp.reshape(-1, ce), colp.reshape(-1, ce))
               for ce in set(_LAP_CE.values()) | {_CE, _DCE}}

    degp = _deg_kernel(edges2d[_DCE][0])
    degp = degp[:, :, :1]

    x_pad = jnp.pad(x, ((0, _NP - _N), (0, 0)))
    dis, tab = _tca(degp, x_pad)

    h = x_pad
    for li, F in ((0, 128), (1, 64), (2, 32)):
        r2, c2 = edges2d[_LAP_CE[F]]
        accp = _lap_kernels[F](tab, r2, c2)
        args = _gate_params(params["lstm" + str(li)], params["lin" + str(li)])
        outs = _gate_stages[li](h, accp, dis, *args)
        h = outs[0]
        if li < 2:
            tab = outs[1]
    return h
